# baseline (speedup 1.0000x reference)
import jax
import jax.numpy as jnp
from jax import lax
from jax.experimental import pallas as pl
from jax.experimental.pallas import tpu as pltpu
from jax.experimental.pallas import tpu_sc as plsc

B = 16
D = 64
T = 32768
MAXLEN = 4096
NW = 32
BATCH = 128
NB = T // (NW * BATCH)
NSLAB = MAXLEN // BATCH
NFILLS = (B * MAXLEN - T) // BATCH
FPW = NFILLS // NW
LN_EPS = 1e-5


def _rsqrt_vec(xv):
    iv = plsc.bitcast(xv, jnp.int32)
    yv = plsc.bitcast(jnp.int32(0x5F3759DF) - (iv >> 1), jnp.float32)
    for _ in range(3):
        yv = yv * (1.5 - 0.5 * xv * yv * yv)
    return yv


def _sc_body(ids_hbm, dslab_hbm, fslab_hbm, item_hbm, ptab_hbm,
             lnw_hbm, lnb_hbm, out_hbm,
             idx_i, idx_d, idx_f, lnw_v, lnb_v,
             pack_v, pos_v, outr_v, bias_v,
             sem_g, sem_p, sem_s, sem_f):
    wid = lax.axis_index("s") * 2 + lax.axis_index("c")

    pltpu.sync_copy(lnw_hbm, lnw_v)
    pltpu.sync_copy(lnb_hbm, lnb_v)
    pltpu.sync_copy(ids_hbm.at[pl.ds(wid * NB, NB)], idx_i)
    pltpu.sync_copy(dslab_hbm.at[wid], idx_d)
    pltpu.sync_copy(fslab_hbm.at[wid], idx_f)
    wv = [lnw_v[pl.ds(c * 16, 16)] for c in range(4)]
    bv = [lnb_v[pl.ds(c * 16, 16)] for c in range(4)]
    lanes = lax.iota(jnp.int32, 16)

    for c in range(4):
        for u in range(16):
            f = c * 16 + u
            fb = jnp.full((16,), bv[c][u], jnp.float32)
            for q in range(BATCH // 16):
                bias_v[f, pl.ds(q * 16, 16)] = fb

    def slab_dst(s):
        b = s >> 5
        p0 = pl.multiple_of((s & (NSLAB - 1)) << 7, BATCH)
        return out_hbm.at[b, :, pl.ds(p0, BATCH)]

    fv = idx_f[pl.ds(0, 16)]
    for u in range(FPW):
        pltpu.async_copy(bias_v, slab_dst(fv[u]), sem_f)

    dv = idx_d[pl.ds(0, 16)]

    def fire_fetch(i, buf):
        def g_body(g, _):
            iv = idx_i[i, pl.ds(g * 16, 16)]
            for u in range(16):
                rid = iv[u]
                pltpu.async_copy(item_hbm.at[rid],
                                 pack_v.at[buf, g * 16 + u], sem_g)
            return 0
        lax.fori_loop(0, BATCH // 16, g_body, 0)
        s = dv[i]
        p0 = pl.multiple_of((s & (NSLAB - 1)) << 7, BATCH)
        return pltpu.async_copy(ptab_hbm.at[:, pl.ds(p0, BATCH)],
                                pos_v.at[buf], sem_p)

    def drain_fetch(buf):
        pltpu.make_async_copy(item_hbm.at[pl.ds(0, BATCH)],
                              pack_v.at[buf], sem_g).wait()

    def ln_batch(buf):
        bufv = jnp.full((16,), buf, jnp.int32)

        def ln_body(r, _):
            rv = jnp.full((16,), r, jnp.int32)
            pv = [plsc.load_gather(pos_v, [bufv, c * 16 + lanes, rv])
                  for c in range(4)]
            v = [pack_v[buf, r, pl.ds(c * 16, 16)] + pv[c] for c in range(4)]
            s1 = (v[0] + v[1]) + (v[2] + v[3])
            s2 = (v[0] * v[0] + v[1] * v[1]) + (v[2] * v[2] + v[3] * v[3])
            m = jnp.sum(s1) * (1.0 / D)
            var = jnp.sum(s2) * (1.0 / D) - m * m
            rstd = _rsqrt_vec(jnp.full((16,), var + LN_EPS, jnp.float32))
            mv = jnp.full((16,), m, jnp.float32)
            for c in range(4):
                plsc.store_scatter(outr_v, [bufv, c * 16 + lanes, rv],
                                   (v[c] - mv) * rstd * wv[c] + bv[c])
            return 0
        lax.fori_loop(0, BATCH, ln_body, 0)

    pos_descs = {0: fire_fetch(0, 0)}
    scat_descs = {}
    for i in range(NB):
        buf = i & 1
        if i + 1 < NB:
            pos_descs[i + 1] = fire_fetch(i + 1, (i + 1) & 1)
        drain_fetch(buf)
        pos_descs.pop(i).wait()
        if i - 2 in scat_descs:
            scat_descs.pop(i - 2).wait()
        ln_batch(buf)
        scat_descs[i] = pltpu.async_copy(outr_v.at[buf], slab_dst(dv[i]),
                                         sem_s)
    for i in sorted(scat_descs):
        scat_descs[i].wait()

    for u in range(FPW):
        pltpu.make_async_copy(ptab_hbm.at[:, pl.ds(0, BATCH)], bias_v,
                              sem_f).wait()


def _mask_body(len_ref, out_ref):
    ii = lax.broadcasted_iota(jnp.int32, (B, MAXLEN), 1)
    out_ref[...] = ii < len_ref[...]


def kernel(ids, lengths, positions, item_table, pos_table, ln_weight, ln_bias):
    lengths = lengths.astype(jnp.int32)
    csum = jnp.cumsum(lengths)
    tb = jnp.arange(0, T, BATCH, dtype=jnp.int32)
    segb = (tb[:, None] >= csum[None, :]).sum(1).astype(jnp.int32)
    p0b = positions[::BATCH]
    dslab = (segb * NSLAB + p0b // BATCH).reshape(NW, NB)
    dslab = jnp.pad(dslab, ((0, 0), (0, 16 - NB)), mode="edge")

    padcnt = (MAXLEN - lengths) // BATCH
    padcum = jnp.cumsum(padcnt)
    k = jnp.arange(NFILLS, dtype=jnp.int32)
    bk = (k[:, None] >= padcum[None, :]).sum(1).astype(jnp.int32)
    padoff = padcum - padcnt
    fslab = (bk * NSLAB + lengths[bk] // BATCH + (k - padoff[bk]))
    fslab = fslab.astype(jnp.int32).reshape(NW, FPW)
    fslab = jnp.pad(fslab, ((0, 0), (0, 16 - FPW)), mode="edge")

    ids2d = ids.reshape(T // BATCH, BATCH)

    mesh = plsc.VectorSubcoreMesh(core_axis_name="c", subcore_axis_name="s")
    sc_call = pl.kernel(
        _sc_body,
        out_type=jax.ShapeDtypeStruct((B, D, MAXLEN), jnp.float32),
        mesh=mesh,
        compiler_params=pltpu.CompilerParams(
            needs_layout_passes=False, use_tc_tiling_on_sc=True),
        scratch_types=[
            pltpu.VMEM((NB, BATCH), jnp.int32),
            pltpu.VMEM((16,), jnp.int32),
            pltpu.VMEM((16,), jnp.int32),
            pltpu.VMEM((D,), jnp.float32),
            pltpu.VMEM((D,), jnp.float32),
            pltpu.VMEM((2, BATCH, D), jnp.float32),
            pltpu.VMEM((2, D, BATCH), jnp.float32),
            pltpu.VMEM((2, D, BATCH), jnp.float32),
            pltpu.VMEM((D, BATCH), jnp.float32),
            pltpu.SemaphoreType.DMA,
            pltpu.SemaphoreType.DMA,
            pltpu.SemaphoreType.DMA,
            pltpu.SemaphoreType.DMA,
        ],
    )
    out_fm = sc_call(ids2d, dslab, fslab, item_table, pos_table.T,
                     ln_weight, ln_bias)
    padded = out_fm.transpose(0, 2, 1)

    mask = pl.pallas_call(
        _mask_body,
        out_shape=jax.ShapeDtypeStruct((B, MAXLEN), jnp.bool_),
    )(lengths.reshape(B, 1))
    return padded, mask


# speedup vs baseline: 1.0632x; 1.0632x over previous
import jax
import jax.numpy as jnp
from jax import lax
from jax.experimental import pallas as pl
from jax.experimental.pallas import tpu as pltpu
from jax.experimental.pallas import tpu_sc as plsc

B = 16
D = 64
T = 32768
MAXLEN = 4096
NW = 32
BATCH = 128
NB = T // (NW * BATCH)
NSLAB = MAXLEN // BATCH
NFILLS = (B * MAXLEN - T) // BATCH
FPW = NFILLS // NW
LN_EPS = 1e-5


def _rsqrt_vec(xv):
    iv = plsc.bitcast(xv, jnp.int32)
    yv = plsc.bitcast(jnp.int32(0x5F3759DF) - (iv >> 1), jnp.float32)
    for _ in range(3):
        yv = yv * (1.5 - 0.5 * xv * yv * yv)
    return yv


def _sc_body(ids_hbm, dslab_hbm, fslab_hbm, item_hbm, ptab_hbm,
             lnw_hbm, lnb_hbm, out_hbm,
             idx_i, idx_d, idx_f, lnw_v, lnb_v,
             pack_v, pos_v, outr_v, bias_v,
             sem_g, sem_p, sem_s, sem_f):
    wid = lax.axis_index("s") * 2 + lax.axis_index("c")

    pltpu.sync_copy(lnw_hbm, lnw_v)
    pltpu.sync_copy(lnb_hbm, lnb_v)
    pltpu.sync_copy(ids_hbm.at[pl.ds(wid * NB, NB)], idx_i)
    pltpu.sync_copy(dslab_hbm.at[wid], idx_d)
    pltpu.sync_copy(fslab_hbm.at[wid], idx_f)
    wv = [lnw_v[pl.ds(c * 16, 16)] for c in range(4)]
    bv = [lnb_v[pl.ds(c * 16, 16)] for c in range(4)]
    lanes = lax.iota(jnp.int32, 16)

    for c in range(4):
        for u in range(16):
            f = c * 16 + u
            fb = jnp.full((16,), bv[c][u], jnp.float32)
            for q in range(BATCH // 16):
                bias_v[f, pl.ds(q * 16, 16)] = fb

    def slab_dst(s):
        b = s >> 5
        p0 = pl.multiple_of((s & (NSLAB - 1)) << 7, BATCH)
        return out_hbm.at[b, :, pl.ds(p0, BATCH)]

    fv = idx_f[pl.ds(0, 16)]
    for u in range(FPW):
        pltpu.async_copy(bias_v, slab_dst(fv[u]), sem_f)

    dv = idx_d[pl.ds(0, 16)]

    def fire_fetch(i, buf):
        def g_body(g, _):
            iv = idx_i[i, pl.ds(g * 16, 16)]
            for u in range(16):
                rid = iv[u]
                pltpu.async_copy(item_hbm.at[rid],
                                 pack_v.at[buf, g * 16 + u], sem_g)
            return 0
        lax.fori_loop(0, BATCH // 16, g_body, 0)
        s = dv[i]
        p0 = pl.multiple_of((s & (NSLAB - 1)) << 7, BATCH)
        return pltpu.async_copy(ptab_hbm.at[pl.ds(p0, BATCH)],
                                pos_v.at[buf], sem_p)

    def drain_fetch(buf):
        pltpu.make_async_copy(item_hbm.at[pl.ds(0, BATCH)],
                              pack_v.at[buf], sem_g).wait()

    def ln_batch(buf):
        bufv = jnp.full((16,), buf, jnp.int32)

        def ln_body(r, _):
            rv = jnp.full((16,), r, jnp.int32)
            v = [pack_v[buf, r, pl.ds(c * 16, 16)]
                 + pos_v[buf, r, pl.ds(c * 16, 16)] for c in range(4)]
            s1 = (v[0] + v[1]) + (v[2] + v[3])
            s2 = (v[0] * v[0] + v[1] * v[1]) + (v[2] * v[2] + v[3] * v[3])
            m = jnp.sum(s1) * (1.0 / D)
            var = jnp.sum(s2) * (1.0 / D) - m * m
            rstd = _rsqrt_vec(jnp.full((16,), var + LN_EPS, jnp.float32))
            mv = jnp.full((16,), m, jnp.float32)
            for c in range(4):
                plsc.store_scatter(outr_v, [bufv, c * 16 + lanes, rv],
                                   (v[c] - mv) * rstd * wv[c] + bv[c])
            return 0
        lax.fori_loop(0, BATCH, ln_body, 0)

    pos_descs = {0: fire_fetch(0, 0)}
    scat_descs = {}
    for i in range(NB):
        buf = i & 1
        if i + 1 < NB:
            pos_descs[i + 1] = fire_fetch(i + 1, (i + 1) & 1)
        drain_fetch(buf)
        pos_descs.pop(i).wait()
        if i - 2 in scat_descs:
            scat_descs.pop(i - 2).wait()
        ln_batch(buf)
        scat_descs[i] = pltpu.async_copy(outr_v.at[buf], slab_dst(dv[i]),
                                         sem_s)
    for i in sorted(scat_descs):
        scat_descs[i].wait()

    for u in range(FPW):
        pltpu.make_async_copy(item_hbm.at[pl.ds(0, D)], bias_v,
                              sem_f).wait()


def _mask_body(len_ref, out_ref):
    ii = lax.broadcasted_iota(jnp.int32, (B, MAXLEN), 1)
    out_ref[...] = ii < len_ref[...]


def kernel(ids, lengths, positions, item_table, pos_table, ln_weight, ln_bias):
    lengths = lengths.astype(jnp.int32)
    csum = jnp.cumsum(lengths)
    tb = jnp.arange(0, T, BATCH, dtype=jnp.int32)
    segb = (tb[:, None] >= csum[None, :]).sum(1).astype(jnp.int32)
    p0b = positions[::BATCH]
    dslab = (segb * NSLAB + p0b // BATCH).reshape(NW, NB)
    dslab = jnp.pad(dslab, ((0, 0), (0, 16 - NB)), mode="edge")

    padcnt = (MAXLEN - lengths) // BATCH
    padcum = jnp.cumsum(padcnt)
    k = jnp.arange(NFILLS, dtype=jnp.int32)
    bk = (k[:, None] >= padcum[None, :]).sum(1).astype(jnp.int32)
    padoff = padcum - padcnt
    fslab = (bk * NSLAB + lengths[bk] // BATCH + (k - padoff[bk]))
    fslab = fslab.astype(jnp.int32).reshape(NW, FPW)
    fslab = jnp.pad(fslab, ((0, 0), (0, 16 - FPW)), mode="edge")

    ids2d = ids.reshape(T // BATCH, BATCH)

    mesh = plsc.VectorSubcoreMesh(core_axis_name="c", subcore_axis_name="s")
    sc_call = pl.kernel(
        _sc_body,
        out_type=jax.ShapeDtypeStruct((B, D, MAXLEN), jnp.float32),
        mesh=mesh,
        compiler_params=pltpu.CompilerParams(
            needs_layout_passes=False, use_tc_tiling_on_sc=True),
        scratch_types=[
            pltpu.VMEM((NB, BATCH), jnp.int32),
            pltpu.VMEM((16,), jnp.int32),
            pltpu.VMEM((16,), jnp.int32),
            pltpu.VMEM((D,), jnp.float32),
            pltpu.VMEM((D,), jnp.float32),
            pltpu.VMEM((2, BATCH, D), jnp.float32),
            pltpu.VMEM((2, BATCH, D), jnp.float32),
            pltpu.VMEM((2, D, BATCH), jnp.float32),
            pltpu.VMEM((D, BATCH), jnp.float32),
            pltpu.SemaphoreType.DMA,
            pltpu.SemaphoreType.DMA,
            pltpu.SemaphoreType.DMA,
            pltpu.SemaphoreType.DMA,
        ],
    )
    out_fm = sc_call(ids2d, dslab, fslab, item_table, pos_table,
                     ln_weight, ln_bias)
    padded = out_fm.transpose(0, 2, 1)

    mask = pl.pallas_call(
        _mask_body,
        out_shape=jax.ShapeDtypeStruct((B, MAXLEN), jnp.bool_),
    )(lengths.reshape(B, 1))
    return padded, mask


# speedup vs baseline: 1.1710x; 1.1014x over previous
import jax
import jax.numpy as jnp
from jax import lax
from jax.experimental import pallas as pl
from jax.experimental.pallas import tpu as pltpu
from jax.experimental.pallas import tpu_sc as plsc

B = 16
D = 64
T = 32768
MAXLEN = 4096
NW = 32
BATCH = 128
NB = T // (NW * BATCH)
G8 = 8
NGRP = T // G8
NFILLG = (B * MAXLEN - T) // G8
FPW = NFILLG // NW
LN_EPS = 1e-5


def _rsqrt_vec(xv):
    iv = plsc.bitcast(xv, jnp.int32)
    yv = plsc.bitcast(jnp.int32(0x5F3759DF) - (iv >> 1), jnp.float32)
    for _ in range(3):
        yv = yv * (1.5 - 0.5 * xv * yv * yv)
    return yv


def _sc_body(ids_hbm, pos_hbm, dst8_hbm, fill8_hbm, item_hbm, ptab_hbm,
             lnw_hbm, lnb_hbm, out_hbm,
             idx_i, idx_p, idx_d, idx_f, lnw_v, lnb_v,
             pack_v, posr_v, outr_v, bias_v,
             sem_g, sem_p, sem_s, sem_f):
    wid = lax.axis_index("s") * 2 + lax.axis_index("c")

    pltpu.sync_copy(lnw_hbm, lnw_v)
    pltpu.sync_copy(lnb_hbm, lnb_v)
    pltpu.sync_copy(ids_hbm.at[pl.ds(wid * NB, NB)], idx_i)
    pltpu.sync_copy(pos_hbm.at[pl.ds(wid * NB, NB)], idx_p)
    pltpu.sync_copy(dst8_hbm.at[wid], idx_d)
    pltpu.sync_copy(fill8_hbm.at[wid], idx_f)
    wv = [lnw_v[pl.ds(c * 16, 16)] for c in range(4)]
    bv = [lnb_v[pl.ds(c * 16, 16)] for c in range(4)]

    for r in range(G8):
        for c in range(4):
            bias_v[r, pl.ds(c * 16, 16)] = bv[c]

    def fill_body(g, _):
        fv = idx_f[pl.ds(g * 16, 16)]
        for u in range(16):
            d8 = fv[u]
            pltpu.async_copy(bias_v, out_hbm.at[pl.ds(d8 * G8, G8)], sem_f)
        return 0
    lax.fori_loop(0, FPW // 16, fill_body, 0)

    def fire_fetch(i, buf):
        def g_body(g, _):
            iv = idx_i[i, pl.ds(g * 16, 16)]
            for u in range(16):
                rid = iv[u]
                pltpu.async_copy(item_hbm.at[rid],
                                 pack_v.at[buf, g * 16 + u], sem_g)
            return 0
        lax.fori_loop(0, BATCH // 16, g_body, 0)
        p0 = pl.multiple_of(idx_p[i, pl.ds(0, 16)][0], BATCH)
        return pltpu.async_copy(ptab_hbm.at[pl.ds(p0, BATCH)],
                                posr_v.at[buf], sem_p)

    def drain_fetch(buf):
        pltpu.make_async_copy(item_hbm.at[pl.ds(0, BATCH)],
                              pack_v.at[buf], sem_g).wait()

    def ln_batch(buf):
        def ln_body(r, _):
            v = [pack_v[buf, r, pl.ds(c * 16, 16)]
                 + posr_v[buf, r, pl.ds(c * 16, 16)] for c in range(4)]
            s1 = (v[0] + v[1]) + (v[2] + v[3])
            s2 = (v[0] * v[0] + v[1] * v[1]) + (v[2] * v[2] + v[3] * v[3])
            m = jnp.sum(s1) * (1.0 / D)
            var = jnp.sum(s2) * (1.0 / D) - m * m
            rstd = _rsqrt_vec(jnp.full((16,), var + LN_EPS, jnp.float32))
            mv = jnp.full((16,), m, jnp.float32)
            for c in range(4):
                outr_v[buf, r, pl.ds(c * 16, 16)] = \
                    (v[c] - mv) * rstd * wv[c] + bv[c]
            return 0
        lax.fori_loop(0, BATCH, ln_body, 0)

    pos_descs = {0: fire_fetch(0, 0)}
    scat_descs = {}
    for i in range(NB):
        buf = i & 1
        if i + 1 < NB:
            pos_descs[i + 1] = fire_fetch(i + 1, (i + 1) & 1)
        drain_fetch(buf)
        pos_descs.pop(i).wait()
        if i - 2 in scat_descs:
            for dsc in scat_descs.pop(i - 2):
                dsc.wait()
        ln_batch(buf)
        dv = idx_d[pl.ds(i * 16, 16)]
        descs = []
        for u in range(16):
            d8 = dv[u]
            descs.append(pltpu.async_copy(
                outr_v.at[buf, pl.ds(u * G8, G8)],
                out_hbm.at[pl.ds(d8 * G8, G8)], sem_s))
        scat_descs[i] = descs
    for i in sorted(scat_descs):
        for dsc in scat_descs[i]:
            dsc.wait()

    def fdrain_body(g, _):
        pltpu.make_async_copy(item_hbm.at[pl.ds(0, G8)], bias_v,
                              sem_f).wait()
        return 0
    lax.fori_loop(0, FPW, fdrain_body, 0)


def _mask_body(len_ref, out_ref):
    ii = lax.broadcasted_iota(jnp.int32, (B, MAXLEN), 1)
    out_ref[...] = ii < len_ref[...]


def kernel(ids, lengths, positions, item_table, pos_table, ln_weight, ln_bias):
    lengths = lengths.astype(jnp.int32)
    csum = jnp.cumsum(lengths)
    t8 = jnp.arange(0, T, G8, dtype=jnp.int32)
    seg8 = (t8[:, None] >= csum[None, :]).sum(1).astype(jnp.int32)
    p8 = positions[::G8]
    dst8 = (seg8 * (MAXLEN // G8) + p8 // G8).reshape(NW, NGRP // NW)

    padcnt = (MAXLEN - lengths) // G8
    padcum = jnp.cumsum(padcnt)
    k = jnp.arange(NFILLG, dtype=jnp.int32)
    bk = (k[:, None] >= padcum[None, :]).sum(1).astype(jnp.int32)
    padoff = padcum - padcnt
    fill8 = (bk * (MAXLEN // G8) + lengths[bk] // G8 + (k - padoff[bk]))
    fill8 = fill8.astype(jnp.int32).reshape(NW, NFILLG // NW)

    ids2d = ids.reshape(T // BATCH, BATCH)
    pos2d = positions.reshape(T // BATCH, BATCH)

    mesh = plsc.VectorSubcoreMesh(core_axis_name="c", subcore_axis_name="s")
    sc_call = pl.kernel(
        _sc_body,
        out_type=jax.ShapeDtypeStruct((B * MAXLEN, D), jnp.float32),
        mesh=mesh,
        compiler_params=pltpu.CompilerParams(
            needs_layout_passes=False, use_tc_tiling_on_sc=True),
        scratch_types=[
            pltpu.VMEM((NB, BATCH), jnp.int32),
            pltpu.VMEM((NB, BATCH), jnp.int32),
            pltpu.VMEM((NGRP // NW,), jnp.int32),
            pltpu.VMEM((NFILLG // NW,), jnp.int32),
            pltpu.VMEM((D,), jnp.float32),
            pltpu.VMEM((D,), jnp.float32),
            pltpu.VMEM((2, BATCH, D), jnp.float32),
            pltpu.VMEM((2, BATCH, D), jnp.float32),
            pltpu.VMEM((2, BATCH, D), jnp.float32),
            pltpu.VMEM((G8, D), jnp.float32),
            pltpu.SemaphoreType.DMA,
            pltpu.SemaphoreType.DMA,
            pltpu.SemaphoreType.DMA,
            pltpu.SemaphoreType.DMA,
        ],
    )
    padded2d = sc_call(ids2d, pos2d, dst8, fill8, item_table, pos_table,
                       ln_weight, ln_bias)
    padded = padded2d.reshape(B, MAXLEN, D)

    mask = pl.pallas_call(
        _mask_body,
        out_shape=jax.ShapeDtypeStruct((B, MAXLEN), jnp.bool_),
    )(lengths.reshape(B, 1))
    return padded, mask
